# baseline (speedup 1.0000x reference)
import functools

import jax
import jax.numpy as jnp
from jax import lax
from jax.experimental import pallas as pl
from jax.experimental.pallas import tpu as pltpu
from jax.experimental.pallas import tpu_sc as plsc

B = 1024
D = 4096
R = 5
NC = 2
NS = 16
L = 16
NW = NC * NS
RPW = B // NW
UN = 16
VB = 8
NGRP = RPW // VB
NBUF = 2
ZB = 8
ZROWS = (R - 1) * B // NW
NZD = ZROWS // ZB

_mesh = plsc.VectorSubcoreMesh(core_axis_name="c", subcore_axis_name="s")


@functools.partial(
    pl.kernel,
    mesh=_mesh,
    out_type=(
        jax.ShapeDtypeStruct((R, B, D), jnp.float32),
        jax.ShapeDtypeStruct((B,), jnp.float32),
    ),
    scratch_types=[
        pltpu.VMEM((RPW + L,), jnp.float32),
        pltpu.VMEM((RPW + L,), jnp.float32),
        pltpu.VMEM((NBUF, 1, VB, D), jnp.float32),
        pltpu.VMEM((1, ZB, D), jnp.float32),
        pltpu.VMEM((RPW,), jnp.float32),
        pltpu.SemaphoreType.DMA,
        pltpu.SemaphoreType.DMA,
        pltpu.SemaphoreType.DMA,
    ],
)
def _sc_body(b_hbm, s_hbm, mats_hbm, ones_hbm,
             bvs, svs, pbuf, zbuf, obuf, semv0, semv1, semz):
    wid = lax.axis_index("s") * NC + lax.axis_index("c")
    base = wid * RPW
    fi = lax.broadcasted_iota(jnp.int32, (L,), 0).astype(jnp.float32)
    zero = jnp.zeros((L,), jnp.float32)
    one = jnp.ones((L,), jnp.float32)
    semv = (semv0, semv1)

    pltpu.sync_copy(b_hbm.at[pl.ds(base, RPW)], bvs.at[pl.ds(0, RPW)])
    pltpu.sync_copy(s_hbm.at[pl.ds(base, RPW)], svs.at[pl.ds(0, RPW)])

    def zb(c, carry):
        for j in range(ZB):
            zbuf[0, j, pl.ds(c * L, L)] = zero
        return carry

    lax.fori_loop(0, D // L, zb, 0)

    for g in range(RPW // L):
        obuf[pl.ds(g * L, L)] = one

    zplane = 1 + wid // (NW // (R - 1))
    zrow0 = (wid % (NW // (R - 1))) * ZROWS
    for i in range(NZD):
        pltpu.async_copy(
            zbuf,
            mats_hbm.at[pl.ds(zplane, 1), pl.ds(zrow0 + i * ZB, ZB)],
            semz,
        )

    def fill_row(par, j, r):
        bb = jnp.full((L,), bvs[pl.ds(r, L)][0], jnp.float32)
        ss = jnp.full((L,), svs[pl.ds(r, L)][0], jnp.float32)
        v0 = bb + fi * ss
        deltas = [ss * jnp.float32(L * k) for k in range(UN)]
        stride = ss * jnp.float32(L * UN)

        def chunk(c, v):
            off = c * (L * UN)
            for k in range(UN):
                pbuf[par, 0, j, pl.ds(off + k * L, L)] = v + deltas[k]
            return v + stride

        lax.fori_loop(0, D // (L * UN), chunk, v0)

    def blk_body(g, carry):
        for par in range(NBUF):
            gi = g * NBUF + par
            r0 = base + gi * VB

            @pl.when(g > 0)
            def _drain():
                pltpu.make_async_copy(
                    pbuf.at[par],
                    mats_hbm.at[pl.ds(0, 1), pl.ds(r0, VB)],
                    semv[par],
                ).wait()

            for j in range(VB):
                fill_row(par, j, gi * VB + j)
            pltpu.async_copy(
                pbuf.at[par],
                mats_hbm.at[pl.ds(0, 1), pl.ds(r0, VB)],
                semv[par],
            )
        return carry

    lax.fori_loop(0, NGRP // NBUF, blk_body, 0)

    for par in range(NBUF):
        last_r0 = base + (NGRP - NBUF + par) * VB
        pltpu.make_async_copy(
            pbuf.at[par],
            mats_hbm.at[pl.ds(0, 1), pl.ds(last_r0, VB)],
            semv[par],
        ).wait()
    for i in range(NZD):
        pltpu.make_async_copy(
            zbuf,
            mats_hbm.at[pl.ds(zplane, 1), pl.ds(zrow0 + i * ZB, ZB)],
            semz,
        ).wait()

    pltpu.sync_copy(obuf, ones_hbm.at[pl.ds(base, RPW)])


def kernel(floats):
    b_arr = floats[:, 0]
    s_arr = (floats[:, 1] - floats[:, 0]) / jnp.float32(D - 1)
    mats5, ones_flat = _sc_body(b_arr, s_arr)
    return (jnp.transpose(mats5, (1, 0, 2)), ones_flat.reshape(B, 1))


# speedup vs baseline: 1.0197x; 1.0197x over previous
import functools

import jax
import jax.numpy as jnp
from jax import lax
from jax.experimental import pallas as pl
from jax.experimental.pallas import tpu as pltpu
from jax.experimental.pallas import tpu_sc as plsc

B = 1024
D = 4096
R = 5
NC = 2
NS = 16
L = 16
NW = NC * NS
RPW = B // NW
UN = 8
VB = 8
NGRP = RPW // VB
NBUF = 2
ZB = 8
ZROWS = (R - 1) * B // NW
NZD = ZROWS // ZB

_mesh = plsc.VectorSubcoreMesh(core_axis_name="c", subcore_axis_name="s")


@functools.partial(
    pl.kernel,
    mesh=_mesh,
    out_type=(
        jax.ShapeDtypeStruct((R, B, D), jnp.float32),
        jax.ShapeDtypeStruct((B,), jnp.float32),
    ),
    scratch_types=[
        pltpu.VMEM((RPW + L,), jnp.float32),
        pltpu.VMEM((RPW + L,), jnp.float32),
        pltpu.VMEM((NBUF, 1, VB, D), jnp.float32),
        pltpu.VMEM((1, ZB, D), jnp.float32),
        pltpu.VMEM((RPW,), jnp.float32),
        pltpu.SemaphoreType.DMA,
        pltpu.SemaphoreType.DMA,
        pltpu.SemaphoreType.DMA,
    ],
)
def _sc_body(b_hbm, s_hbm, mats_hbm, ones_hbm,
             bvs, svs, pbuf, zbuf, obuf, semv0, semv1, semz):
    wid = lax.axis_index("s") * NC + lax.axis_index("c")
    base = wid * RPW
    fi = lax.broadcasted_iota(jnp.int32, (L,), 0).astype(jnp.float32)
    zero = jnp.zeros((L,), jnp.float32)
    one = jnp.ones((L,), jnp.float32)
    semv = (semv0, semv1)

    pltpu.sync_copy(b_hbm.at[pl.ds(base, RPW)], bvs.at[pl.ds(0, RPW)])
    pltpu.sync_copy(s_hbm.at[pl.ds(base, RPW)], svs.at[pl.ds(0, RPW)])

    def zb(c, carry):
        for j in range(ZB):
            zbuf[0, j, pl.ds(c * L, L)] = zero
        return carry

    lax.fori_loop(0, D // L, zb, 0)

    for g in range(RPW // L):
        obuf[pl.ds(g * L, L)] = one

    zplane = 1 + wid // (NW // (R - 1))
    zrow0 = (wid % (NW // (R - 1))) * ZROWS
    for i in range(NZD):
        pltpu.async_copy(
            zbuf,
            mats_hbm.at[pl.ds(zplane, 1), pl.ds(zrow0 + i * ZB, ZB)],
            semz,
        )

    def fill_row(par, j, r):
        bb = jnp.full((L,), bvs[pl.ds(r, L)][0], jnp.float32)
        ss = jnp.full((L,), svs[pl.ds(r, L)][0], jnp.float32)
        v0 = bb + fi * ss
        deltas = [ss * jnp.float32(L * k) for k in range(UN)]
        stride = ss * jnp.float32(L * UN)

        def chunk(c, v):
            off = c * (L * UN)
            for k in range(UN):
                pbuf[par, 0, j, pl.ds(off + k * L, L)] = v + deltas[k]
            return v + stride

        lax.fori_loop(0, D // (L * UN), chunk, v0)

    def blk_body(g, carry):
        for par in range(NBUF):
            gi = g * NBUF + par
            r0 = base + gi * VB

            @pl.when(g > 0)
            def _drain():
                pltpu.make_async_copy(
                    pbuf.at[par],
                    mats_hbm.at[pl.ds(0, 1), pl.ds(r0, VB)],
                    semv[par],
                ).wait()

            for j in range(VB):
                fill_row(par, j, gi * VB + j)
            pltpu.async_copy(
                pbuf.at[par],
                mats_hbm.at[pl.ds(0, 1), pl.ds(r0, VB)],
                semv[par],
            )
        return carry

    lax.fori_loop(0, NGRP // NBUF, blk_body, 0)

    for par in range(NBUF):
        last_r0 = base + (NGRP - NBUF + par) * VB
        pltpu.make_async_copy(
            pbuf.at[par],
            mats_hbm.at[pl.ds(0, 1), pl.ds(last_r0, VB)],
            semv[par],
        ).wait()
    for i in range(NZD):
        pltpu.make_async_copy(
            zbuf,
            mats_hbm.at[pl.ds(zplane, 1), pl.ds(zrow0 + i * ZB, ZB)],
            semz,
        ).wait()

    pltpu.sync_copy(obuf, ones_hbm.at[pl.ds(base, RPW)])


def kernel(floats):
    b_arr = floats[:, 0]
    s_arr = (floats[:, 1] - floats[:, 0]) / jnp.float32(D - 1)
    mats5, ones_flat = _sc_body(b_arr, s_arr)
    return (jnp.transpose(mats5, (1, 0, 2)), ones_flat.reshape(B, 1))
